# baseline (speedup 1.0000x reference)
import functools

import jax
import jax.numpy as jnp
import numpy as np
from jax import lax
from jax.experimental import pallas as pl
from jax.experimental.pallas import tpu as pltpu
from jax.experimental.pallas import tpu_sc as plsc

B = 8
C_IN = 3
N = 2048
K = 20
C_OUT = 16
BN = B * N
R = B * N * K
KC = K * C_OUT
EPS = 1e-5

TM = 256
RB = 1024
NEG = float("-inf")


def _leaky(v):
    return jnp.where(v >= 0, v, 0.01 * v)


def _a0_body(xt_ref, xb_ref, w1t_ref, w3_ref, u1_ref, y3tT_ref, st1_ref):
    i = pl.program_id(0)
    xt = xt_ref[...]
    u1 = jnp.dot(xt, w1t_ref[...], preferred_element_type=jnp.float32)
    u1_ref[...] = u1
    y3tT_ref[...] = jnp.dot(w3_ref[...], xb_ref[0],
                            preferred_element_type=jnp.float32)

    @pl.when(i == 0)
    def _():
        st1_ref[...] = jnp.zeros_like(st1_ref)

    s = jnp.sum(u1, axis=0, keepdims=True)
    ss = jnp.sum(u1 * u1, axis=0, keepdims=True)
    st1_ref[...] += jnp.concatenate([s, ss], axis=0)


def _a1_body(u1_ref, st1_ref, w2t_ref, g1_ref, b1_ref, u2_ref, st2_ref):
    i = pl.program_id(0)
    st = st1_ref[...]
    m = st[0:1, :] * (1.0 / BN)
    v = st[1:2, :] * (1.0 / BN) - m * m
    a = g1_ref[...] / jnp.sqrt(v + EPS)
    c = b1_ref[...] - m * a
    h1 = _leaky(u1_ref[...] * a + c)
    u2 = jnp.dot(h1, w2t_ref[...], preferred_element_type=jnp.float32)
    u2_ref[...] = u2

    @pl.when(i == 0)
    def _():
        st2_ref[...] = jnp.zeros_like(st2_ref)

    s = jnp.sum(u2)
    ss = jnp.sum(u2 * u2)
    st2_ref[...] += jnp.concatenate(
        [s.reshape(1, 1), ss.reshape(1, 1)], axis=1)


def _knn_body(xt_ref, xb_ref, idx_ref):
    b = pl.program_id(0)
    xr = xt_ref[...]
    xc = xb_ref[0]
    inner = lax.dot_general(
        xr, xc, (((1,), (0,)), ((), ())),
        preferred_element_type=jnp.float32)
    xx_r = jnp.sum(xr * xr, axis=1, keepdims=True)
    xx_c = jnp.sum(xc * xc, axis=0, keepdims=True)
    d = 2.0 * inner - xx_r - xx_c
    col = lax.broadcasted_iota(jnp.int32, (TM, N), 1)
    kcol = lax.broadcasted_iota(jnp.int32, (TM, K), 1)
    acc = jnp.zeros((TM, K), jnp.int32)
    for t in range(K):
        am = jnp.argmax(d, axis=1).astype(jnp.int32)[:, None]
        acc = acc + jnp.where(kcol == t, am, 0)
        d = jnp.where(col == am, NEG, d)
    idx_ref[0] = acc


NW = 32
WPB = NW // B
ROWS_PER_W = R // NW
BN_PER_W = BN // NW
CHB = 16
NCHB = BN_PER_W // CHB


def _gather_sc(table_hbm, gidx_hbm, out_hbm, tab_v, idx_v, rows_v,
               sem_t, sem_i, sem_o0, sem_o1, *, wpb=WPB, rpw=ROWS_PER_W,
               bnpw=BN_PER_W, nchb=NCHB, b_off=0):
    c = lax.axis_index("c")
    s = lax.axis_index("s")
    wid = s * 2 + c
    b = b_off + wid // wpb
    bn_base = wid * bnpw
    gbase = wid * rpw
    ct = pltpu.async_copy(table_hbm.at[:, pl.ds(b * N, N)], tab_v, sem_t)
    ci = pltpu.async_copy(gidx_hbm.at[pl.ds(gbase, rpw)], idx_v, sem_i)
    ct.wait()
    ci.wait()
    lane = lax.broadcasted_iota(jnp.int32, (16,), 0)

    def do_chunk(j, slot):
        slot_ids = jnp.full((16,), slot, jnp.int32)

        def grp(g, carry):
            rid = g * 16 + lane
            bn_l = rid // K
            kc0 = (rid % K) * C_OUT
            nbr = idx_v[pl.ds(j * (CHB * K) + g * 16, 16)]
            for col in range(C_OUT):
                vals = plsc.load_gather(
                    tab_v, [jnp.full((16,), col, jnp.int32), nbr])
                plsc.store_scatter(rows_v, [slot_ids, bn_l, kc0 + col], vals)
            return carry

        lax.fori_loop(0, CHB * K // 16, grp, 0)

    def out_copy(j, slot, sem):
        return pltpu.make_async_copy(
            rows_v.at[slot], out_hbm.at[pl.ds(bn_base + j * CHB, CHB), :], sem)

    def body(jj, carry):
        for slot, sem in ((0, sem_o0), (1, sem_o1)):
            j = jj * 2 + slot

            @pl.when(jj > 0)
            def _():
                out_copy(j - 2, slot, sem).wait()

            do_chunk(j, slot)
            out_copy(j, slot, sem).start()
        return carry

    lax.fori_loop(0, nchb // 2, body, 0)
    out_copy(nchb - 2, 0, sem_o0).wait()
    out_copy(nchb - 1, 1, sem_o1).wait()


def _e0_body(xlp_ref, st3_ref):
    i = pl.program_id(0)
    v = xlp_ref[...]
    s = jnp.sum(v, axis=0, keepdims=True)
    ss = jnp.sum(v * v, axis=0, keepdims=True)

    @pl.when(i == 0)
    def _():
        st3_ref[...] = jnp.zeros_like(st3_ref)

    st3_ref[...] += jnp.concatenate([s, ss], axis=0)


def _e1_body(xlp_ref, st3_ref, sel_ref, selt_ref, w4b_ref, g3_ref, b3_ref,
             xlout_ref, xk2_ref, st4_ref):
    i = pl.program_id(0)
    cs = st3_ref[...]
    sel = sel_ref[...]
    m16 = jnp.dot(cs[0:1, :], sel, preferred_element_type=jnp.float32) * (1.0 / R)
    q16 = jnp.dot(cs[1:2, :], sel, preferred_element_type=jnp.float32) * (1.0 / R)
    v16 = q16 - m16 * m16
    a16 = g3_ref[...] / jnp.sqrt(v16 + EPS)
    c16 = b3_ref[...] - m16 * a16
    selt = selt_ref[...]
    a320 = jnp.dot(a16, selt, preferred_element_type=jnp.float32)
    c320 = jnp.dot(c16, selt, preferred_element_type=jnp.float32)
    xl = _leaky(xlp_ref[...] * a320 + c320)
    xlout_ref[...] = xl
    xk2 = jnp.dot(xl, w4b_ref[...], preferred_element_type=jnp.float32)
    xk2_ref[...] = xk2

    @pl.when(i == 0)
    def _():
        st4_ref[...] = jnp.zeros_like(st4_ref)

    s = jnp.sum(xk2)
    ss = jnp.sum(xk2 * xk2)
    st4_ref[...] += jnp.concatenate(
        [s.reshape(1, 1), ss.reshape(1, 1)], axis=1)


def _e2_body(xlp_ref, xk2_ref, u2_ref, st2_ref, st4_ref, g2_ref, b2_ref,
             g4_ref, b4_ref, exp_ref, sel_ref, feat_ref):
    st2 = st2_ref[...]
    m2 = st2[0:1, 0:1] * (1.0 / BN)
    v2 = st2[0:1, 1:2] * (1.0 / BN) - m2 * m2
    a2 = g2_ref[...] / jnp.sqrt(v2 + EPS)
    c2 = b2_ref[...] - m2 * a2
    xg = _leaky(u2_ref[...] * a2 + c2)

    st4 = st4_ref[...]
    m4 = st4[0:1, 0:1] * (1.0 / R)
    v4 = st4[0:1, 1:2] * (1.0 / R) - m4 * m4
    a4 = g4_ref[...] / jnp.sqrt(v4 + EPS)
    c4 = b4_ref[...] - m4 * a4
    xk2 = _leaky(xk2_ref[...] * a4 + c4)

    comb = _leaky(xg + xk2)
    mx = jnp.max(comb, axis=1, keepdims=True)
    e = jnp.exp(comb - mx)
    alpha = e / jnp.sum(e, axis=1, keepdims=True)
    aexp = jnp.dot(alpha, exp_ref[...], preferred_element_type=jnp.float32)
    feat_ref[...] = jnp.dot(aexp * xlp_ref[...], sel_ref[...],
                            preferred_element_type=jnp.float32)


def kernel(x, W1, g1, b1, W2, g2, b2, W3, g3, b3, W4, g4, b4):
    f32 = jnp.float32
    xt = jnp.swapaxes(x, 1, 2).reshape(BN, C_IN)

    sel_np = np.zeros((KC, C_OUT), np.float32)
    for k in range(K):
        sel_np[k * C_OUT:(k + 1) * C_OUT, :] = np.eye(C_OUT, dtype=np.float32)
    exp_np = np.zeros((K, KC), np.float32)
    for k in range(K):
        exp_np[k, k * C_OUT:(k + 1) * C_OUT] = 1.0
    SEL = jnp.asarray(sel_np)
    SELT = jnp.asarray(sel_np.T)
    EXPM = jnp.asarray(exp_np)
    w4blk = jnp.asarray(exp_np.T) * jnp.tile(W4[0], K)[:, None]

    nrb = BN // RB
    u1, y3tT, st1 = pl.pallas_call(
        _a0_body,
        grid=(B,),
        in_specs=[
            pl.BlockSpec((N, C_IN), lambda i: (i, 0)),
            pl.BlockSpec((1, C_IN, N), lambda i: (i, 0, 0)),
            pl.BlockSpec((C_IN, C_OUT), lambda i: (0, 0)),
            pl.BlockSpec((C_OUT, C_IN), lambda i: (0, 0)),
        ],
        out_specs=[
            pl.BlockSpec((N, C_OUT), lambda i: (i, 0)),
            pl.BlockSpec((C_OUT, N), lambda i: (0, i)),
            pl.BlockSpec((2, C_OUT), lambda i: (0, 0)),
        ],
        out_shape=[
            jax.ShapeDtypeStruct((BN, C_OUT), f32),
            jax.ShapeDtypeStruct((C_OUT, BN), f32),
            jax.ShapeDtypeStruct((2, C_OUT), f32),
        ],
    )(xt, x, W1.T, W3)

    u2, st2 = pl.pallas_call(
        _a1_body,
        grid=(nrb,),
        in_specs=[
            pl.BlockSpec((RB, C_OUT), lambda i: (i, 0)),
            pl.BlockSpec((2, C_OUT), lambda i: (0, 0)),
            pl.BlockSpec((C_OUT, 1), lambda i: (0, 0)),
            pl.BlockSpec((1, C_OUT), lambda i: (0, 0)),
            pl.BlockSpec((1, C_OUT), lambda i: (0, 0)),
        ],
        out_specs=[
            pl.BlockSpec((RB, 1), lambda i: (i, 0)),
            pl.BlockSpec((1, 2), lambda i: (0, 0)),
        ],
        out_shape=[
            jax.ShapeDtypeStruct((BN, 1), f32),
            jax.ShapeDtypeStruct((1, 2), f32),
        ],
    )(u1, st1, W2.T, g1.reshape(1, C_OUT), b1.reshape(1, C_OUT))

    ntb = N // TM
    HB = B // 2
    mesh = plsc.VectorSubcoreMesh(core_axis_name="c", subcore_axis_name="s")
    rpw_h = (R // 2) // NW
    bnpw_h = (BN // 2) // NW
    xlp_halves = []
    for h in range(2):
        gidx_h = pl.pallas_call(
            _knn_body,
            grid=(HB, ntb),
            in_specs=[
                pl.BlockSpec((TM, C_IN),
                             lambda b, i, h=h: ((h * HB + b) * ntb + i, 0)),
                pl.BlockSpec((1, C_IN, N), lambda b, i, h=h: (h * HB + b, 0, 0)),
            ],
            out_specs=pl.BlockSpec((1, TM, K), lambda b, i: (b, i, 0)),
            out_shape=jax.ShapeDtypeStruct((HB, N, K), jnp.int32),
        )(xt, x)
        xlp_h = pl.kernel(
            functools.partial(_gather_sc, wpb=NW // HB, rpw=rpw_h,
                              bnpw=bnpw_h, nchb=bnpw_h // CHB, b_off=h * HB),
            mesh=mesh,
            compiler_params=pltpu.CompilerParams(needs_layout_passes=False),
            out_type=jax.ShapeDtypeStruct((BN // 2, KC), f32),
            scratch_types=[
                pltpu.VMEM((C_OUT, N), f32),
                pltpu.VMEM((rpw_h,), jnp.int32),
                pltpu.VMEM((2, CHB, KC), f32),
                pltpu.SemaphoreType.DMA,
                pltpu.SemaphoreType.DMA,
                pltpu.SemaphoreType.DMA,
                pltpu.SemaphoreType.DMA,
            ],
        )(y3tT, gidx_h.reshape(R // 2))
        xlp_halves.append(xlp_h)
    xlp2 = jnp.concatenate(xlp_halves, axis=0)

    st3 = pl.pallas_call(
        _e0_body,
        grid=(nrb,),
        in_specs=[pl.BlockSpec((RB, KC), lambda i: (i, 0))],
        out_specs=pl.BlockSpec((2, KC), lambda i: (0, 0)),
        out_shape=jax.ShapeDtypeStruct((2, KC), f32),
    )(xlp2)

    xl_p, xk2p, st4 = pl.pallas_call(
        _e1_body,
        grid=(nrb,),
        in_specs=[
            pl.BlockSpec((RB, KC), lambda i: (i, 0)),
            pl.BlockSpec((2, KC), lambda i: (0, 0)),
            pl.BlockSpec((KC, C_OUT), lambda i: (0, 0)),
            pl.BlockSpec((C_OUT, KC), lambda i: (0, 0)),
            pl.BlockSpec((KC, K), lambda i: (0, 0)),
            pl.BlockSpec((1, C_OUT), lambda i: (0, 0)),
            pl.BlockSpec((1, C_OUT), lambda i: (0, 0)),
        ],
        out_specs=[
            pl.BlockSpec((RB, KC), lambda i: (i, 0)),
            pl.BlockSpec((RB, K), lambda i: (i, 0)),
            pl.BlockSpec((1, 2), lambda i: (0, 0)),
        ],
        out_shape=[
            jax.ShapeDtypeStruct((BN, KC), f32),
            jax.ShapeDtypeStruct((BN, K), f32),
            jax.ShapeDtypeStruct((1, 2), f32),
        ],
    )(xlp2, st3, SEL, SELT, w4blk, g3.reshape(1, C_OUT), b3.reshape(1, C_OUT))

    x_feat = pl.pallas_call(
        _e2_body,
        grid=(nrb,),
        in_specs=[
            pl.BlockSpec((RB, KC), lambda i: (i, 0)),
            pl.BlockSpec((RB, K), lambda i: (i, 0)),
            pl.BlockSpec((RB, 1), lambda i: (i, 0)),
            pl.BlockSpec((1, 2), lambda i: (0, 0)),
            pl.BlockSpec((1, 2), lambda i: (0, 0)),
            pl.BlockSpec((1, 1), lambda i: (0, 0)),
            pl.BlockSpec((1, 1), lambda i: (0, 0)),
            pl.BlockSpec((1, 1), lambda i: (0, 0)),
            pl.BlockSpec((1, 1), lambda i: (0, 0)),
            pl.BlockSpec((K, KC), lambda i: (0, 0)),
            pl.BlockSpec((KC, C_OUT), lambda i: (0, 0)),
        ],
        out_specs=pl.BlockSpec((RB, C_OUT), lambda i: (i, 0)),
        out_shape=jax.ShapeDtypeStruct((BN, C_OUT), f32),
    )(xl_p, xk2p, u2, st2, st4, g2.reshape(1, 1), b2.reshape(1, 1),
      g4.reshape(1, 1), b4.reshape(1, 1), EXPM, SEL)

    return (x_feat.reshape(B, N, C_OUT), xl_p.reshape(B, N, K, C_OUT))


# speedup vs baseline: 1.0042x; 1.0042x over previous
import functools

import jax
import jax.numpy as jnp
import numpy as np
from jax import lax
from jax.experimental import pallas as pl
from jax.experimental.pallas import tpu as pltpu
from jax.experimental.pallas import tpu_sc as plsc

B = 8
C_IN = 3
N = 2048
K = 20
C_OUT = 16
BN = B * N
R = B * N * K
KC = K * C_OUT
EPS = 1e-5

TM = 256
RB = 1024
NEG = float("-inf")


def _leaky(v):
    return jnp.where(v >= 0, v, 0.01 * v)


def _knn_body(xt_ref, xb_ref, w1t_ref, w3_ref, idx_ref, y3tT_ref, u1_ref,
              st1_ref):
    b = pl.program_id(0)
    i = pl.program_id(1)
    xr = xt_ref[...]
    xc = xb_ref[0]

    @pl.when(i == 0)
    def _():
        y3tT_ref[...] = jnp.dot(w3_ref[...], xc,
                                preferred_element_type=jnp.float32)

    u1b = jnp.dot(xr, w1t_ref[...], preferred_element_type=jnp.float32)
    u1_ref[...] = u1b

    @pl.when(jnp.logical_and(b == 0, i == 0))
    def _():
        st1_ref[...] = jnp.zeros_like(st1_ref)

    st1_ref[...] += jnp.concatenate(
        [jnp.sum(u1b, axis=0, keepdims=True),
         jnp.sum(u1b * u1b, axis=0, keepdims=True)], axis=0)
    inner = lax.dot_general(
        xr, xc, (((1,), (0,)), ((), ())),
        preferred_element_type=jnp.float32)
    xx_r = jnp.sum(xr * xr, axis=1, keepdims=True)
    xx_c = jnp.sum(xc * xc, axis=0, keepdims=True)
    d = 2.0 * inner - xx_r - xx_c
    col = lax.broadcasted_iota(jnp.int32, (TM, N), 1)
    kcol = lax.broadcasted_iota(jnp.int32, (TM, K), 1)
    acc = jnp.zeros((TM, K), jnp.int32)
    for t in range(K):
        am = jnp.argmax(d, axis=1).astype(jnp.int32)[:, None]
        acc = acc + jnp.where(kcol == t, am, 0)
        d = jnp.where(col == am, NEG, d)
    idx_ref[0] = acc


NW = 32
WPB = NW // B
ROWS_PER_W = R // NW
BN_PER_W = BN // NW
CHB = 16
NCHB = BN_PER_W // CHB


def _gather_sc(table_hbm, gidx_hbm, out_hbm, tab_v, idx_v, rows_v,
               sem_t, sem_i, sem_o0, sem_o1, *, wpb=WPB, rpw=ROWS_PER_W,
               bnpw=BN_PER_W, nchb=NCHB, b_off=0):
    c = lax.axis_index("c")
    s = lax.axis_index("s")
    wid = s * 2 + c
    b = b_off + wid // wpb
    bn_base = wid * bnpw
    gbase = wid * rpw
    ct = pltpu.async_copy(table_hbm.at[:, pl.ds(b * N, N)], tab_v, sem_t)
    ci = pltpu.async_copy(gidx_hbm.at[pl.ds(gbase, rpw)], idx_v, sem_i)
    ct.wait()
    ci.wait()
    lane = lax.broadcasted_iota(jnp.int32, (16,), 0)

    def do_chunk(j, slot):
        slot_ids = jnp.full((16,), slot, jnp.int32)

        def grp(g, carry):
            rid = g * 16 + lane
            bn_l = rid // K
            kc0 = (rid % K) * C_OUT
            nbr = idx_v[pl.ds(j * (CHB * K) + g * 16, 16)]
            for col in range(C_OUT):
                vals = plsc.load_gather(
                    tab_v, [jnp.full((16,), col, jnp.int32), nbr])
                plsc.store_scatter(rows_v, [slot_ids, bn_l, kc0 + col], vals)
            return carry

        lax.fori_loop(0, CHB * K // 16, grp, 0)

    def out_copy(j, slot, sem):
        return pltpu.make_async_copy(
            rows_v.at[slot], out_hbm.at[pl.ds(bn_base + j * CHB, CHB), :], sem)

    def body(jj, carry):
        for slot, sem in ((0, sem_o0), (1, sem_o1)):
            j = jj * 2 + slot

            @pl.when(jj > 0)
            def _():
                out_copy(j - 2, slot, sem).wait()

            do_chunk(j, slot)
            out_copy(j, slot, sem).start()
        return carry

    lax.fori_loop(0, nchb // 2, body, 0)
    out_copy(nchb - 2, 0, sem_o0).wait()
    out_copy(nchb - 1, 1, sem_o1).wait()


def _a1e0_body(u1_ref, st1_ref, w2t_ref, g1_ref, b1_ref, xlp_ref,
               u2_ref, st2_ref, st3_ref):
    i = pl.program_id(0)
    st = st1_ref[...]
    m = st[0:1, :] * (1.0 / BN)
    v = st[1:2, :] * (1.0 / BN) - m * m
    a = g1_ref[...] / jnp.sqrt(v + EPS)
    c = b1_ref[...] - m * a
    h1 = _leaky(u1_ref[...] * a + c)
    u2 = jnp.dot(h1, w2t_ref[...], preferred_element_type=jnp.float32)
    u2_ref[...] = u2

    vv = xlp_ref[...]

    @pl.when(i == 0)
    def _():
        st2_ref[...] = jnp.zeros_like(st2_ref)
        st3_ref[...] = jnp.zeros_like(st3_ref)

    st2_ref[...] += jnp.concatenate(
        [jnp.sum(u2).reshape(1, 1), jnp.sum(u2 * u2).reshape(1, 1)], axis=1)
    st3_ref[...] += jnp.concatenate(
        [jnp.sum(vv, axis=0, keepdims=True),
         jnp.sum(vv * vv, axis=0, keepdims=True)], axis=0)


def _e1_body(xlp_ref, st3_ref, sel_ref, selt_ref, w4b_ref, g3_ref, b3_ref,
             xlout_ref, xk2_ref, st4_ref):
    i = pl.program_id(0)
    cs = st3_ref[...]
    sel = sel_ref[...]
    m16 = jnp.dot(cs[0:1, :], sel, preferred_element_type=jnp.float32) * (1.0 / R)
    q16 = jnp.dot(cs[1:2, :], sel, preferred_element_type=jnp.float32) * (1.0 / R)
    v16 = q16 - m16 * m16
    a16 = g3_ref[...] / jnp.sqrt(v16 + EPS)
    c16 = b3_ref[...] - m16 * a16
    selt = selt_ref[...]
    a320 = jnp.dot(a16, selt, preferred_element_type=jnp.float32)
    c320 = jnp.dot(c16, selt, preferred_element_type=jnp.float32)
    xl = _leaky(xlp_ref[...] * a320 + c320)
    xlout_ref[...] = xl
    xk2 = jnp.dot(xl, w4b_ref[...], preferred_element_type=jnp.float32)
    xk2_ref[...] = xk2

    @pl.when(i == 0)
    def _():
        st4_ref[...] = jnp.zeros_like(st4_ref)

    s = jnp.sum(xk2)
    ss = jnp.sum(xk2 * xk2)
    st4_ref[...] += jnp.concatenate(
        [s.reshape(1, 1), ss.reshape(1, 1)], axis=1)


def _e2_body(xlp_ref, xk2_ref, u2_ref, st2_ref, st4_ref, g2_ref, b2_ref,
             g4_ref, b4_ref, exp_ref, sel_ref, feat_ref):
    st2 = st2_ref[...]
    m2 = st2[0:1, 0:1] * (1.0 / BN)
    v2 = st2[0:1, 1:2] * (1.0 / BN) - m2 * m2
    a2 = g2_ref[...] / jnp.sqrt(v2 + EPS)
    c2 = b2_ref[...] - m2 * a2
    xg = _leaky(u2_ref[...] * a2 + c2)

    st4 = st4_ref[...]
    m4 = st4[0:1, 0:1] * (1.0 / R)
    v4 = st4[0:1, 1:2] * (1.0 / R) - m4 * m4
    a4 = g4_ref[...] / jnp.sqrt(v4 + EPS)
    c4 = b4_ref[...] - m4 * a4
    xk2 = _leaky(xk2_ref[...] * a4 + c4)

    comb = _leaky(xg + xk2)
    mx = jnp.max(comb, axis=1, keepdims=True)
    e = jnp.exp(comb - mx)
    alpha = e / jnp.sum(e, axis=1, keepdims=True)
    aexp = jnp.dot(alpha, exp_ref[...], preferred_element_type=jnp.float32)
    feat_ref[...] = jnp.dot(aexp * xlp_ref[...], sel_ref[...],
                            preferred_element_type=jnp.float32)


def kernel(x, W1, g1, b1, W2, g2, b2, W3, g3, b3, W4, g4, b4):
    f32 = jnp.float32
    xt = jnp.swapaxes(x, 1, 2).reshape(BN, C_IN)

    sel_np = np.zeros((KC, C_OUT), np.float32)
    for k in range(K):
        sel_np[k * C_OUT:(k + 1) * C_OUT, :] = np.eye(C_OUT, dtype=np.float32)
    exp_np = np.zeros((K, KC), np.float32)
    for k in range(K):
        exp_np[k, k * C_OUT:(k + 1) * C_OUT] = 1.0
    SEL = jnp.asarray(sel_np)
    SELT = jnp.asarray(sel_np.T)
    EXPM = jnp.asarray(exp_np)
    w4blk = jnp.asarray(exp_np.T) * jnp.tile(W4[0], K)[:, None]

    nrb = BN // RB
    ntb = N // TM
    gidx, y3tT, u1, st1 = pl.pallas_call(
        _knn_body,
        grid=(B, ntb),
        in_specs=[
            pl.BlockSpec((TM, C_IN), lambda b, i: (b * ntb + i, 0)),
            pl.BlockSpec((1, C_IN, N), lambda b, i: (b, 0, 0)),
            pl.BlockSpec((C_IN, C_OUT), lambda b, i: (0, 0)),
            pl.BlockSpec((C_OUT, C_IN), lambda b, i: (0, 0)),
        ],
        out_specs=[
            pl.BlockSpec((1, TM, K), lambda b, i: (b, i, 0)),
            pl.BlockSpec((C_OUT, N), lambda b, i: (0, b)),
            pl.BlockSpec((TM, C_OUT), lambda b, i: (b * ntb + i, 0)),
            pl.BlockSpec((2, C_OUT), lambda b, i: (0, 0)),
        ],
        out_shape=[
            jax.ShapeDtypeStruct((B, N, K), jnp.int32),
            jax.ShapeDtypeStruct((C_OUT, BN), f32),
            jax.ShapeDtypeStruct((BN, C_OUT), f32),
            jax.ShapeDtypeStruct((2, C_OUT), f32),
        ],
    )(xt, x, W1.T, W3)

    mesh = plsc.VectorSubcoreMesh(core_axis_name="c", subcore_axis_name="s")
    xlp2 = pl.kernel(
        _gather_sc,
        mesh=mesh,
        compiler_params=pltpu.CompilerParams(needs_layout_passes=False),
        out_type=jax.ShapeDtypeStruct((BN, KC), f32),
        scratch_types=[
            pltpu.VMEM((C_OUT, N), f32),
            pltpu.VMEM((ROWS_PER_W,), jnp.int32),
            pltpu.VMEM((2, CHB, KC), f32),
            pltpu.SemaphoreType.DMA,
            pltpu.SemaphoreType.DMA,
            pltpu.SemaphoreType.DMA,
            pltpu.SemaphoreType.DMA,
        ],
    )(y3tT, gidx.reshape(R))

    u2, st2, st3 = pl.pallas_call(
        _a1e0_body,
        grid=(nrb,),
        in_specs=[
            pl.BlockSpec((RB, C_OUT), lambda i: (i, 0)),
            pl.BlockSpec((2, C_OUT), lambda i: (0, 0)),
            pl.BlockSpec((C_OUT, 1), lambda i: (0, 0)),
            pl.BlockSpec((1, C_OUT), lambda i: (0, 0)),
            pl.BlockSpec((1, C_OUT), lambda i: (0, 0)),
            pl.BlockSpec((RB, KC), lambda i: (i, 0)),
        ],
        out_specs=[
            pl.BlockSpec((RB, 1), lambda i: (i, 0)),
            pl.BlockSpec((1, 2), lambda i: (0, 0)),
            pl.BlockSpec((2, KC), lambda i: (0, 0)),
        ],
        out_shape=[
            jax.ShapeDtypeStruct((BN, 1), f32),
            jax.ShapeDtypeStruct((1, 2), f32),
            jax.ShapeDtypeStruct((2, KC), f32),
        ],
    )(u1, st1, W2.T, g1.reshape(1, C_OUT), b1.reshape(1, C_OUT), xlp2)

    xl_p, xk2p, st4 = pl.pallas_call(
        _e1_body,
        grid=(nrb,),
        in_specs=[
            pl.BlockSpec((RB, KC), lambda i: (i, 0)),
            pl.BlockSpec((2, KC), lambda i: (0, 0)),
            pl.BlockSpec((KC, C_OUT), lambda i: (0, 0)),
            pl.BlockSpec((C_OUT, KC), lambda i: (0, 0)),
            pl.BlockSpec((KC, K), lambda i: (0, 0)),
            pl.BlockSpec((1, C_OUT), lambda i: (0, 0)),
            pl.BlockSpec((1, C_OUT), lambda i: (0, 0)),
        ],
        out_specs=[
            pl.BlockSpec((RB, KC), lambda i: (i, 0)),
            pl.BlockSpec((RB, K), lambda i: (i, 0)),
            pl.BlockSpec((1, 2), lambda i: (0, 0)),
        ],
        out_shape=[
            jax.ShapeDtypeStruct((BN, KC), f32),
            jax.ShapeDtypeStruct((BN, K), f32),
            jax.ShapeDtypeStruct((1, 2), f32),
        ],
    )(xlp2, st3, SEL, SELT, w4blk, g3.reshape(1, C_OUT), b3.reshape(1, C_OUT))

    x_feat = pl.pallas_call(
        _e2_body,
        grid=(nrb,),
        in_specs=[
            pl.BlockSpec((RB, KC), lambda i: (i, 0)),
            pl.BlockSpec((RB, K), lambda i: (i, 0)),
            pl.BlockSpec((RB, 1), lambda i: (i, 0)),
            pl.BlockSpec((1, 2), lambda i: (0, 0)),
            pl.BlockSpec((1, 2), lambda i: (0, 0)),
            pl.BlockSpec((1, 1), lambda i: (0, 0)),
            pl.BlockSpec((1, 1), lambda i: (0, 0)),
            pl.BlockSpec((1, 1), lambda i: (0, 0)),
            pl.BlockSpec((1, 1), lambda i: (0, 0)),
            pl.BlockSpec((K, KC), lambda i: (0, 0)),
            pl.BlockSpec((KC, C_OUT), lambda i: (0, 0)),
        ],
        out_specs=pl.BlockSpec((RB, C_OUT), lambda i: (i, 0)),
        out_shape=jax.ShapeDtypeStruct((BN, C_OUT), f32),
    )(xl_p, xk2p, u2, st2, st4, g2.reshape(1, 1), b2.reshape(1, 1),
      g4.reshape(1, 1), b4.reshape(1, 1), EXPM, SEL)

    return (x_feat.reshape(B, N, C_OUT), xl_p.reshape(B, N, K, C_OUT))
